# baseline (speedup 1.0000x reference)
import functools

import jax
import jax.numpy as jnp
from jax import lax
from jax.experimental import pallas as pl
from jax.experimental.pallas import tpu as pltpu
from jax.experimental.pallas import tpu_sc as plsc

_NC = 2
_NS = 16
_L = 16
_NW = _NC * _NS
_B = 16384
_BPW = _B // _NW
_F = 64
_CH = 128
_NCH = _BPW // _CH
_HALF = _BPW // 2

_mesh = plsc.VectorSubcoreMesh(core_axis_name="c", subcore_axis_name="s")


@functools.partial(
    pl.kernel,
    mesh=_mesh,
    out_type=jax.ShapeDtypeStruct((_B,), jnp.float32),
    compiler_params=pltpu.CompilerParams(
        needs_layout_passes=False, use_tc_tiling_on_sc=True
    ),
    scratch_types=[
        pltpu.VMEM((_NCH, _CH), jnp.int32),
        pltpu.VMEM((_NCH, _CH), jnp.int32),
        pltpu.VMEM((_NCH, _CH), jnp.int32),
        pltpu.VMEM((_NCH, _CH), jnp.int32),
        pltpu.VMEM((_BPW,), jnp.int32),
        pltpu.VMEM((_BPW,), jnp.int32),
        pltpu.VMEM((_HALF, 2 * _F), jnp.float32),
        pltpu.VMEM((_HALF, 2 * _F), jnp.float32),
        pltpu.VMEM((_L, _L), jnp.float32),
        pltpu.VMEM((_BPW,), jnp.float32),
        pltpu.SemaphoreType.DMA,
    ],
)
def _dots_sc(uids_hbm, iids_hbm, ufac_hbm, ifac_hbm, out_hbm, uidx, iidx,
             ubrow, ibrow, uoff, ioff, upad, ipad, tile, outv, sem):
    c = lax.axis_index("c")
    s = lax.axis_index("s")
    wid = s * _NC + c
    base = wid * _BPW

    for j in range(_NCH):
        pltpu.sync_copy(uids_hbm.at[pl.ds(base + j * _CH, _CH)], uidx.at[j])
        pltpu.sync_copy(iids_hbm.at[pl.ds(base + j * _CH, _CH)], iidx.at[j])

    for j in range(_NCH):
        for t in range(_CH // _L):
            sl = pl.ds(t * _L, _L)
            fl = pl.ds(j * _CH + t * _L, _L)
            uv = uidx[j, sl]
            iv = iidx[j, sl]
            um = uv < _HU
            im = iv < _HI
            ubrow[j, sl] = jnp.where(um, uv, uv - _HU)
            ibrow[j, sl] = jnp.where(im, iv, iv - _HI)
            uoff[fl] = jnp.where(um, 0, _F).astype(jnp.int32)
            ioff[fl] = jnp.where(im, 0, _F).astype(jnp.int32)

    lane = lax.iota(jnp.int32, _L)
    zeros = jnp.zeros((_L,), jnp.int32)

    for h in range(2):
        copies = []
        for jj in range(2):
            j = 2 * h + jj
            dst = pl.ds(jj * _CH, _CH)
            copies.append(pltpu.async_copy(ufac_hbm.at[ubrow.at[j]], upad.at[dst], sem))
            copies.append(pltpu.async_copy(ifac_hbm.at[ibrow.at[j]], ipad.at[dst], sem))
        for cp in copies:
            cp.wait()

        def group_body(g, carry):
            row0 = g * _L
            uo16 = uoff[pl.ds(h * _HALF + row0, _L)]
            io16 = ioff[pl.ds(h * _HALF + row0, _L)]
            for r in range(_L):
                lr = row0 + r
                bu = uo16[r]
                bi = io16[r]
                acc = upad[lr, pl.ds(bu, _L)] * ipad[lr, pl.ds(bi, _L)]
                for k in range(1, _F // _L):
                    acc = acc + (upad[lr, pl.ds(bu + k * _L, _L)]
                                 * ipad[lr, pl.ds(bi + k * _L, _L)])
                tile[r, ...] = acc
            ssum = plsc.load_gather(tile, [lane, zeros])
            for j in range(1, _L):
                ssum = ssum + plsc.load_gather(tile, [lane, jnp.full((_L,), j, jnp.int32)])
            outv[pl.ds(h * _HALF + row0, _L)] = ssum
            return carry

        lax.fori_loop(0, _HALF // _L, group_body, 0)

    pltpu.sync_copy(outv, out_hbm.at[pl.ds(base, _BPW)])


@functools.partial(
    pl.kernel,
    mesh=_mesh,
    out_type=jax.ShapeDtypeStruct((_B,), jnp.float32),
    compiler_params=pltpu.CompilerParams(
        needs_layout_passes=False, use_tc_tiling_on_sc=False
    ),
    scratch_types=[
        pltpu.VMEM((_NCH, _CH), jnp.int32),
        pltpu.VMEM((_NCH, _CH), jnp.int32),
        pltpu.VMEM((_NCH, _CH), jnp.int32),
        pltpu.VMEM((_NCH, _CH), jnp.int32),
        pltpu.VMEM((_BPW,), jnp.int32),
        pltpu.VMEM((_BPW,), jnp.int32),
        pltpu.VMEM((_BPW, _L), jnp.float32),
        pltpu.VMEM((_BPW, _L), jnp.float32),
        pltpu.VMEM((_L,), jnp.float32),
        pltpu.VMEM((_BPW,), jnp.float32),
        pltpu.VMEM((_BPW,), jnp.float32),
        pltpu.SemaphoreType.DMA,
    ],
)
def _bias_sc(uids_hbm, iids_hbm, ubias_hbm, ibias_hbm, gb_hbm, part_hbm,
             out_hbm, uidx, iidx, ubidx, ibidx, ulo, ilo, ubrows, ibrows,
             gbv, pv, outv, sem):
    c = lax.axis_index("c")
    s = lax.axis_index("s")
    wid = s * _NC + c
    base = wid * _BPW

    pltpu.sync_copy(uids_hbm.at[pl.ds(wid * _NCH, _NCH)], uidx)
    pltpu.sync_copy(iids_hbm.at[pl.ds(wid * _NCH, _NCH)], iidx)
    pltpu.sync_copy(gb_hbm, gbv)
    pltpu.sync_copy(part_hbm.at[pl.ds(base, _BPW)], pv)

    for j in range(_NCH):
        for t in range(_CH // _L):
            sl = pl.ds(t * _L, _L)
            fl = pl.ds(j * _CH + t * _L, _L)
            uv = uidx[j, sl]
            iv = iidx[j, sl]
            ubidx[j, sl] = jnp.right_shift(uv, 4)
            ibidx[j, sl] = jnp.right_shift(iv, 4)
            ulo[fl] = jnp.bitwise_and(uv, 15)
            ilo[fl] = jnp.bitwise_and(iv, 15)

    copies = []
    for j in range(_NCH):
        dst = pl.ds(j * _CH, _CH)
        copies.append(pltpu.async_copy(ubias_hbm.at[ubidx.at[j]], ubrows.at[dst], sem))
        copies.append(pltpu.async_copy(ibias_hbm.at[ibidx.at[j]], ibrows.at[dst], sem))
    for cp in copies:
        cp.wait()

    lane = lax.iota(jnp.int32, _L)
    gb = gbv[...]

    def group_body(g, carry):
        row0 = g * _L
        rows16 = row0 + lane
        ub = plsc.load_gather(ubrows, [rows16, ulo[pl.ds(row0, _L)]])
        ib = plsc.load_gather(ibrows, [rows16, ilo[pl.ds(row0, _L)]])
        outv[pl.ds(row0, _L)] = pv[pl.ds(row0, _L)] + ub + ib + gb
        return carry

    lax.fori_loop(0, _BPW // _L, group_body, 0)

    pltpu.sync_copy(outv, out_hbm.at[pl.ds(base, _BPW)])


_PACKC = 512
_HU = 500224
_HI = 50176


def _pack_body(xa_ref, xb_ref, o_ref):
    ri = lax.broadcasted_iota(jnp.int32, (_F, _F), 0)
    ci = lax.broadcasted_iota(jnp.int32, (_F, _F), 1)
    ident = (ri == ci).astype(jnp.float32)
    dn = (((0,), (0,)), ((), ()))
    o_ref[:, 0:_F] = lax.dot_general(xa_ref[...], ident, dn,
                                     preferred_element_type=jnp.float32)
    o_ref[:, _F:2 * _F] = lax.dot_general(xb_ref[...], ident, dn,
                                          preferred_element_type=jnp.float32)


def _pack(table_t, half):
    hb = half // _PACKC
    return pl.pallas_call(
        _pack_body,
        grid=(hb,),
        in_specs=[
            pl.BlockSpec((_F, _PACKC), lambda j: (0, j)),
            pl.BlockSpec((_F, _PACKC), lambda j: (0, j + hb)),
        ],
        out_specs=pl.BlockSpec((_PACKC, 2 * _F), lambda j: (j, 0)),
        out_shape=jax.ShapeDtypeStruct((half, 2 * _F), jnp.float32),
    )(table_t, table_t)


def kernel(user_ids, item_ids, user_factors, item_factors, user_bias,
           item_bias, global_bias):
    uf2 = _pack(user_factors.T, _HU)
    if2 = _pack(item_factors.T, _HI)
    part = _dots_sc(user_ids, item_ids, uf2, if2)
    uids2 = user_ids.reshape(_B // _CH, _CH)
    iids2 = item_ids.reshape(_B // _CH, _CH)
    ub2 = user_bias.reshape(user_bias.shape[0] // _L, _L)
    ib2 = item_bias.reshape(item_bias.shape[0] // _L, _L)
    gb16 = jnp.broadcast_to(global_bias.astype(jnp.float32).reshape(()), (_L,))
    return _bias_sc(uids2, iids2, ub2, ib2, gb16, part)


# speedup vs baseline: 2.3323x; 2.3323x over previous
import functools

import jax
import jax.numpy as jnp
from jax import lax
from jax.experimental import pallas as pl
from jax.experimental.pallas import tpu as pltpu
from jax.experimental.pallas import tpu_sc as plsc

_NC = 2
_NS = 16
_L = 16
_NW = _NC * _NS
_B = 16384
_BPW = _B // _NW
_F = 64
_CH = 128
_NCH = _BPW // _CH
_HALF = _BPW // 2

_mesh = plsc.VectorSubcoreMesh(core_axis_name="c", subcore_axis_name="s")


@functools.partial(
    pl.kernel,
    mesh=_mesh,
    out_type=jax.ShapeDtypeStruct((_B,), jnp.float32),
    compiler_params=pltpu.CompilerParams(
        needs_layout_passes=False, use_tc_tiling_on_sc=True
    ),
    scratch_types=[
        pltpu.VMEM((_NCH, _CH), jnp.int32),
        pltpu.VMEM((_NCH, _CH), jnp.int32),
        pltpu.VMEM((_NCH, _CH), jnp.int32),
        pltpu.VMEM((_NCH, _CH), jnp.int32),
        pltpu.VMEM((_BPW,), jnp.int32),
        pltpu.VMEM((_BPW,), jnp.int32),
        pltpu.VMEM((_HALF, 2 * _F), jnp.float32),
        pltpu.VMEM((_HALF, 2 * _F), jnp.float32),
        pltpu.VMEM((_L, _L), jnp.float32),
        pltpu.VMEM((_BPW,), jnp.float32),
        pltpu.SemaphoreType.DMA,
    ],
)
def _dots_sc(uids_hbm, iids_hbm, ufac_hbm, ifac_hbm, out_hbm, uidx, iidx,
             ubrow, ibrow, uoff, ioff, upad, ipad, tile, outv, sem):
    c = lax.axis_index("c")
    s = lax.axis_index("s")
    wid = s * _NC + c
    base = wid * _BPW

    for j in range(_NCH):
        pltpu.sync_copy(uids_hbm.at[pl.ds(base + j * _CH, _CH)], uidx.at[j])
        pltpu.sync_copy(iids_hbm.at[pl.ds(base + j * _CH, _CH)], iidx.at[j])

    for j in range(_NCH):
        for t in range(_CH // _L):
            sl = pl.ds(t * _L, _L)
            fl = pl.ds(j * _CH + t * _L, _L)
            uv = uidx[j, sl]
            iv = iidx[j, sl]
            um = uv < _HU
            im = iv < _HI
            ubrow[j, sl] = jnp.where(um, uv, uv - _HU)
            ibrow[j, sl] = jnp.where(im, iv, iv - _HI)
            uoff[fl] = jnp.where(um, 0, _F).astype(jnp.int32)
            ioff[fl] = jnp.where(im, 0, _F).astype(jnp.int32)

    lane = lax.iota(jnp.int32, _L)
    zeros = jnp.zeros((_L,), jnp.int32)

    for h in range(2):
        copies = []
        for jj in range(2):
            j = 2 * h + jj
            dst = pl.ds(jj * _CH, _CH)
            copies.append(pltpu.async_copy(ufac_hbm.at[ubrow.at[j]], upad.at[dst], sem))
            copies.append(pltpu.async_copy(ifac_hbm.at[ibrow.at[j]], ipad.at[dst], sem))
        for cp in copies:
            cp.wait()

        def group_body(g, carry):
            row0 = g * _L
            uo16 = uoff[pl.ds(h * _HALF + row0, _L)]
            io16 = ioff[pl.ds(h * _HALF + row0, _L)]
            for r in range(_L):
                lr = row0 + r
                bu = uo16[r]
                bi = io16[r]
                acc = upad[lr, pl.ds(bu, _L)] * ipad[lr, pl.ds(bi, _L)]
                for k in range(1, _F // _L):
                    acc = acc + (upad[lr, pl.ds(bu + k * _L, _L)]
                                 * ipad[lr, pl.ds(bi + k * _L, _L)])
                tile[r, ...] = acc
            ssum = plsc.load_gather(tile, [lane, zeros])
            for j in range(1, _L):
                ssum = ssum + plsc.load_gather(tile, [lane, jnp.full((_L,), j, jnp.int32)])
            outv[pl.ds(h * _HALF + row0, _L)] = ssum
            return carry

        lax.fori_loop(0, _HALF // _L, group_body, 0)

    pltpu.sync_copy(outv, out_hbm.at[pl.ds(base, _BPW)])


@functools.partial(
    pl.kernel,
    mesh=_mesh,
    out_type=jax.ShapeDtypeStruct((_B,), jnp.float32),
    compiler_params=pltpu.CompilerParams(
        needs_layout_passes=False, use_tc_tiling_on_sc=False
    ),
    scratch_types=[
        pltpu.VMEM((_NCH, _CH), jnp.int32),
        pltpu.VMEM((_NCH, _CH), jnp.int32),
        pltpu.VMEM((_NCH, _CH), jnp.int32),
        pltpu.VMEM((_NCH, _CH), jnp.int32),
        pltpu.VMEM((_BPW,), jnp.int32),
        pltpu.VMEM((_BPW,), jnp.int32),
        pltpu.VMEM((_BPW, _L), jnp.float32),
        pltpu.VMEM((_BPW, _L), jnp.float32),
        pltpu.VMEM((_L,), jnp.float32),
        pltpu.VMEM((_BPW,), jnp.float32),
        pltpu.VMEM((_BPW,), jnp.float32),
        pltpu.SemaphoreType.DMA,
    ],
)
def _bias_sc(uids_hbm, iids_hbm, ubias_hbm, ibias_hbm, gb_hbm, part_hbm,
             out_hbm, uidx, iidx, ubidx, ibidx, ulo, ilo, ubrows, ibrows,
             gbv, pv, outv, sem):
    c = lax.axis_index("c")
    s = lax.axis_index("s")
    wid = s * _NC + c
    base = wid * _BPW

    pltpu.sync_copy(uids_hbm.at[pl.ds(wid * _NCH, _NCH)], uidx)
    pltpu.sync_copy(iids_hbm.at[pl.ds(wid * _NCH, _NCH)], iidx)
    pltpu.sync_copy(gb_hbm, gbv)
    pltpu.sync_copy(part_hbm.at[pl.ds(base, _BPW)], pv)

    for j in range(_NCH):
        for t in range(_CH // _L):
            sl = pl.ds(t * _L, _L)
            fl = pl.ds(j * _CH + t * _L, _L)
            uv = uidx[j, sl]
            iv = iidx[j, sl]
            ubidx[j, sl] = jnp.right_shift(uv, 4)
            ibidx[j, sl] = jnp.right_shift(iv, 4)
            ulo[fl] = jnp.bitwise_and(uv, 15)
            ilo[fl] = jnp.bitwise_and(iv, 15)

    copies = []
    for j in range(_NCH):
        dst = pl.ds(j * _CH, _CH)
        copies.append(pltpu.async_copy(ubias_hbm.at[ubidx.at[j]], ubrows.at[dst], sem))
        copies.append(pltpu.async_copy(ibias_hbm.at[ibidx.at[j]], ibrows.at[dst], sem))
    for cp in copies:
        cp.wait()

    lane = lax.iota(jnp.int32, _L)
    gb = gbv[...]

    def group_body(g, carry):
        row0 = g * _L
        rows16 = row0 + lane
        ub = plsc.load_gather(ubrows, [rows16, ulo[pl.ds(row0, _L)]])
        ib = plsc.load_gather(ibrows, [rows16, ilo[pl.ds(row0, _L)]])
        outv[pl.ds(row0, _L)] = pv[pl.ds(row0, _L)] + ub + ib + gb
        return carry

    lax.fori_loop(0, _BPW // _L, group_body, 0)

    pltpu.sync_copy(outv, out_hbm.at[pl.ds(base, _BPW)])


_PACKC = 512
_HU = 500224
_HI = 50176


_PACKK = 16


def _pack_body(*refs):
    o_ref = refs[-1]
    ri = lax.broadcasted_iota(jnp.int32, (_F, _F), 0)
    ci = lax.broadcasted_iota(jnp.int32, (_F, _F), 1)
    ident = (ri == ci).astype(jnp.float32)
    dn = (((0,), (0,)), ((), ()))
    for k in range(_PACKK):
        rows = pl.ds(k * _PACKC, _PACKC)
        o_ref[rows, 0:_F] = lax.dot_general(
            refs[k][...], ident, dn, preferred_element_type=jnp.float32)
        o_ref[rows, _F:2 * _F] = lax.dot_general(
            refs[_PACKK + k][...], ident, dn,
            preferred_element_type=jnp.float32)


def _pack(table_t, half):
    hb = half // _PACKC
    nbmax = table_t.shape[1] // _PACKC
    in_specs = []
    for k in range(_PACKK):
        in_specs.append(pl.BlockSpec(
            (_F, _PACKC), lambda j, k=k: (0, jnp.minimum(_PACKK * j + k, nbmax))))
    for k in range(_PACKK):
        in_specs.append(pl.BlockSpec(
            (_F, _PACKC),
            lambda j, k=k: (0, jnp.minimum(_PACKK * j + k + hb, nbmax))))
    return pl.pallas_call(
        _pack_body,
        grid=(pl.cdiv(hb, _PACKK),),
        in_specs=in_specs,
        out_specs=pl.BlockSpec((_PACKK * _PACKC, 2 * _F), lambda j: (j, 0)),
        out_shape=jax.ShapeDtypeStruct((half, 2 * _F), jnp.float32),
    )(*([table_t] * (2 * _PACKK)))


def kernel(user_ids, item_ids, user_factors, item_factors, user_bias,
           item_bias, global_bias):
    uf2 = _pack(user_factors.T, _HU)
    if2 = _pack(item_factors.T, _HI)
    part = _dots_sc(user_ids, item_ids, uf2, if2)
    uids2 = user_ids.reshape(_B // _CH, _CH)
    iids2 = item_ids.reshape(_B // _CH, _CH)
    ub2 = user_bias.reshape(user_bias.shape[0] // _L, _L)
    ib2 = item_bias.reshape(item_bias.shape[0] // _L, _L)
    gb16 = jnp.broadcast_to(global_bias.astype(jnp.float32).reshape(()), (_L,))
    return _bias_sc(uids2, iids2, ub2, ib2, gb16, part)


# speedup vs baseline: 2.3945x; 1.0266x over previous
import functools

import jax
import jax.numpy as jnp
from jax import lax
from jax.experimental import pallas as pl
from jax.experimental.pallas import tpu as pltpu
from jax.experimental.pallas import tpu_sc as plsc

_NC = 2
_NS = 16
_L = 16
_NW = _NC * _NS
_B = 16384
_BPW = _B // _NW
_F = 64
_CH = 128
_NCH = _BPW // _CH
_HALF = _BPW // 2

_mesh = plsc.VectorSubcoreMesh(core_axis_name="c", subcore_axis_name="s")


@functools.partial(
    pl.kernel,
    mesh=_mesh,
    out_type=jax.ShapeDtypeStruct((_B,), jnp.float32),
    compiler_params=pltpu.CompilerParams(
        needs_layout_passes=False, use_tc_tiling_on_sc=True
    ),
    scratch_types=[
        pltpu.VMEM((_NCH, _CH), jnp.int32),
        pltpu.VMEM((_NCH, _CH), jnp.int32),
        pltpu.VMEM((_NCH, _CH), jnp.int32),
        pltpu.VMEM((_NCH, _CH), jnp.int32),
        pltpu.VMEM((_BPW,), jnp.int32),
        pltpu.VMEM((_BPW,), jnp.int32),
        pltpu.VMEM((_HALF, 2 * _F), jnp.float32),
        pltpu.VMEM((_HALF, 2 * _F), jnp.float32),
        pltpu.VMEM((_L, _L), jnp.float32),
        pltpu.VMEM((_BPW,), jnp.float32),
        pltpu.VMEM((_BPW,), jnp.float32),
        pltpu.SemaphoreType.DMA,
    ],
)
def _dots_sc(uids_hbm, iids_hbm, ufac_hbm, ifac_hbm, bsum_hbm, out_hbm, uidx,
             iidx, ubrow, ibrow, uoff, ioff, upad, ipad, tile, bv, outv, sem):
    c = lax.axis_index("c")
    s = lax.axis_index("s")
    wid = s * _NC + c
    base = wid * _BPW

    pltpu.sync_copy(bsum_hbm.at[pl.ds(base, _BPW)], bv)
    for j in range(_NCH):
        pltpu.sync_copy(uids_hbm.at[pl.ds(base + j * _CH, _CH)], uidx.at[j])
        pltpu.sync_copy(iids_hbm.at[pl.ds(base + j * _CH, _CH)], iidx.at[j])

    for j in range(_NCH):
        for t in range(_CH // _L):
            sl = pl.ds(t * _L, _L)
            fl = pl.ds(j * _CH + t * _L, _L)
            uv = uidx[j, sl]
            iv = iidx[j, sl]
            um = uv < _HU
            im = iv < _HI
            ubrow[j, sl] = jnp.where(um, uv, uv - _HU)
            ibrow[j, sl] = jnp.where(im, iv, iv - _HI)
            uoff[fl] = jnp.where(um, 0, _F).astype(jnp.int32)
            ioff[fl] = jnp.where(im, 0, _F).astype(jnp.int32)

    lane = lax.iota(jnp.int32, _L)
    zeros = jnp.zeros((_L,), jnp.int32)

    for h in range(2):
        copies = []
        for jj in range(2):
            j = 2 * h + jj
            dst = pl.ds(jj * _CH, _CH)
            copies.append(pltpu.async_copy(ufac_hbm.at[ubrow.at[j]], upad.at[dst], sem))
            copies.append(pltpu.async_copy(ifac_hbm.at[ibrow.at[j]], ipad.at[dst], sem))
        for cp in copies:
            cp.wait()

        def group_body(g, carry):
            row0 = g * _L
            uo16 = uoff[pl.ds(h * _HALF + row0, _L)]
            io16 = ioff[pl.ds(h * _HALF + row0, _L)]
            for r in range(_L):
                lr = row0 + r
                bu = uo16[r]
                bi = io16[r]
                acc = upad[lr, pl.ds(bu, _L)] * ipad[lr, pl.ds(bi, _L)]
                for k in range(1, _F // _L):
                    acc = acc + (upad[lr, pl.ds(bu + k * _L, _L)]
                                 * ipad[lr, pl.ds(bi + k * _L, _L)])
                tile[r, ...] = acc
            ssum = plsc.load_gather(tile, [lane, zeros])
            for j in range(1, _L):
                ssum = ssum + plsc.load_gather(tile, [lane, jnp.full((_L,), j, jnp.int32)])
            gr = pl.ds(h * _HALF + row0, _L)
            outv[gr] = ssum + bv[gr]
            return carry

        lax.fori_loop(0, _HALF // _L, group_body, 0)

    pltpu.sync_copy(outv, out_hbm.at[pl.ds(base, _BPW)])


@functools.partial(
    pl.kernel,
    mesh=_mesh,
    out_type=jax.ShapeDtypeStruct((_B,), jnp.float32),
    compiler_params=pltpu.CompilerParams(
        needs_layout_passes=False, use_tc_tiling_on_sc=False
    ),
    scratch_types=[
        pltpu.VMEM((_NCH, _CH), jnp.int32),
        pltpu.VMEM((_NCH, _CH), jnp.int32),
        pltpu.VMEM((_NCH, _CH), jnp.int32),
        pltpu.VMEM((_NCH, _CH), jnp.int32),
        pltpu.VMEM((_BPW,), jnp.int32),
        pltpu.VMEM((_BPW,), jnp.int32),
        pltpu.VMEM((_BPW, _L), jnp.float32),
        pltpu.VMEM((_BPW, _L), jnp.float32),
        pltpu.VMEM((_L,), jnp.float32),
        pltpu.VMEM((_BPW,), jnp.float32),
        pltpu.SemaphoreType.DMA,
    ],
)
def _bias_sc(uids_hbm, iids_hbm, ubias_hbm, ibias_hbm, gb_hbm,
             out_hbm, uidx, iidx, ubidx, ibidx, ulo, ilo, ubrows, ibrows,
             gbv, outv, sem):
    c = lax.axis_index("c")
    s = lax.axis_index("s")
    wid = s * _NC + c
    base = wid * _BPW

    pltpu.sync_copy(uids_hbm.at[pl.ds(wid * _NCH, _NCH)], uidx)
    pltpu.sync_copy(iids_hbm.at[pl.ds(wid * _NCH, _NCH)], iidx)
    pltpu.sync_copy(gb_hbm, gbv)

    for j in range(_NCH):
        for t in range(_CH // _L):
            sl = pl.ds(t * _L, _L)
            fl = pl.ds(j * _CH + t * _L, _L)
            uv = uidx[j, sl]
            iv = iidx[j, sl]
            ubidx[j, sl] = jnp.right_shift(uv, 4)
            ibidx[j, sl] = jnp.right_shift(iv, 4)
            ulo[fl] = jnp.bitwise_and(uv, 15)
            ilo[fl] = jnp.bitwise_and(iv, 15)

    copies = []
    for j in range(_NCH):
        dst = pl.ds(j * _CH, _CH)
        copies.append(pltpu.async_copy(ubias_hbm.at[ubidx.at[j]], ubrows.at[dst], sem))
        copies.append(pltpu.async_copy(ibias_hbm.at[ibidx.at[j]], ibrows.at[dst], sem))
    for cp in copies:
        cp.wait()

    lane = lax.iota(jnp.int32, _L)
    gb = gbv[...]

    def group_body(g, carry):
        row0 = g * _L
        rows16 = row0 + lane
        ub = plsc.load_gather(ubrows, [rows16, ulo[pl.ds(row0, _L)]])
        ib = plsc.load_gather(ibrows, [rows16, ilo[pl.ds(row0, _L)]])
        outv[pl.ds(row0, _L)] = ub + ib + gb
        return carry

    lax.fori_loop(0, _BPW // _L, group_body, 0)

    pltpu.sync_copy(outv, out_hbm.at[pl.ds(base, _BPW)])


_PACKC = 512
_HU = 500224
_HI = 50176


_PACKK = 16


def _pack_body(*refs):
    xa_ref = refs[0]
    o_ref = refs[-1]
    ri = lax.broadcasted_iota(jnp.int32, (_F, _F), 0)
    ci = lax.broadcasted_iota(jnp.int32, (_F, _F), 1)
    ident = (ri == ci).astype(jnp.float32)
    dn = (((0,), (0,)), ((), ()))
    for k in range(_PACKK):
        rows = pl.ds(k * _PACKC, _PACKC)
        o_ref[rows, 0:_F] = lax.dot_general(
            xa_ref[:, pl.ds(k * _PACKC, _PACKC)], ident, dn,
            preferred_element_type=jnp.float32)
        o_ref[rows, _F:2 * _F] = lax.dot_general(
            refs[1 + k][...], ident, dn,
            preferred_element_type=jnp.float32)


def _pack(table_t, half):
    hb = half // _PACKC
    nbmax = table_t.shape[1] // _PACKC
    in_specs = [pl.BlockSpec((_F, _PACKK * _PACKC), lambda j: (0, j))]
    for k in range(_PACKK):
        in_specs.append(pl.BlockSpec(
            (_F, _PACKC),
            lambda j, k=k: (0, jnp.minimum(_PACKK * j + k + hb, nbmax))))
    return pl.pallas_call(
        _pack_body,
        grid=(pl.cdiv(hb, _PACKK),),
        in_specs=in_specs,
        out_specs=pl.BlockSpec((_PACKK * _PACKC, 2 * _F), lambda j: (j, 0)),
        out_shape=jax.ShapeDtypeStruct((half, 2 * _F), jnp.float32),
    )(*([table_t] * (1 + _PACKK)))


def kernel(user_ids, item_ids, user_factors, item_factors, user_bias,
           item_bias, global_bias):
    uids2 = user_ids.reshape(_B // _CH, _CH)
    iids2 = item_ids.reshape(_B // _CH, _CH)
    ub2 = user_bias.reshape(user_bias.shape[0] // _L, _L)
    ib2 = item_bias.reshape(item_bias.shape[0] // _L, _L)
    gb16 = jnp.broadcast_to(global_bias.astype(jnp.float32).reshape(()), (_L,))
    bsum = _bias_sc(uids2, iids2, ub2, ib2, gb16)
    uf2 = _pack(user_factors.T, _HU)
    if2 = _pack(item_factors.T, _HI)
    return _dots_sc(user_ids, item_ids, uf2, if2, bsum)


# speedup vs baseline: 2.4071x; 1.0053x over previous
import functools

import jax
import jax.numpy as jnp
from jax import lax
from jax.experimental import pallas as pl
from jax.experimental.pallas import tpu as pltpu
from jax.experimental.pallas import tpu_sc as plsc

_NC = 2
_NS = 16
_L = 16
_NW = _NC * _NS
_B = 16384
_BPW = _B // _NW
_F = 64
_CH = 128
_NCH = _BPW // _CH
_HALF = _BPW // 2

_mesh = plsc.VectorSubcoreMesh(core_axis_name="c", subcore_axis_name="s")


@functools.partial(
    pl.kernel,
    mesh=_mesh,
    out_type=jax.ShapeDtypeStruct((_B,), jnp.float32),
    compiler_params=pltpu.CompilerParams(
        needs_layout_passes=False, use_tc_tiling_on_sc=True
    ),
    scratch_types=[
        pltpu.VMEM((_NCH, _CH), jnp.int32),
        pltpu.VMEM((_NCH, _CH), jnp.int32),
        pltpu.VMEM((_NCH, _CH), jnp.int32),
        pltpu.VMEM((_NCH, _CH), jnp.int32),
        pltpu.VMEM((_BPW,), jnp.int32),
        pltpu.VMEM((_BPW,), jnp.int32),
        pltpu.VMEM((_HALF, 2 * _F), jnp.float32),
        pltpu.VMEM((_HALF, 2 * _F), jnp.float32),
        pltpu.VMEM((_L, _L), jnp.float32),
        pltpu.VMEM((_BPW,), jnp.float32),
        pltpu.VMEM((_BPW,), jnp.float32),
        pltpu.SemaphoreType.DMA,
    ],
)
def _dots_sc(uids_hbm, iids_hbm, ufac_hbm, ifac_hbm, bsum_hbm, out_hbm, uidx,
             iidx, ubrow, ibrow, uoff, ioff, upad, ipad, tile, bv, outv, sem):
    c = lax.axis_index("c")
    s = lax.axis_index("s")
    wid = s * _NC + c
    base = wid * _BPW

    pltpu.sync_copy(bsum_hbm.at[pl.ds(base, _BPW)], bv)
    for j in range(_NCH):
        pltpu.sync_copy(uids_hbm.at[pl.ds(base + j * _CH, _CH)], uidx.at[j])
        pltpu.sync_copy(iids_hbm.at[pl.ds(base + j * _CH, _CH)], iidx.at[j])

    for j in range(_NCH):
        for t in range(_CH // _L):
            sl = pl.ds(t * _L, _L)
            fl = pl.ds(j * _CH + t * _L, _L)
            uv = uidx[j, sl]
            iv = iidx[j, sl]
            um = uv < _HU
            im = iv < _HI
            ubrow[j, sl] = jnp.where(um, uv, uv - _HU)
            ibrow[j, sl] = jnp.where(im, iv, iv - _HI)
            uoff[fl] = jnp.where(um, 0, _F).astype(jnp.int32)
            ioff[fl] = jnp.where(im, 0, _F).astype(jnp.int32)

    lane = lax.iota(jnp.int32, _L)
    zeros = jnp.zeros((_L,), jnp.int32)

    for h in range(2):
        copies = []
        for jj in range(2):
            j = 2 * h + jj
            dst = pl.ds(jj * _CH, _CH)
            copies.append(pltpu.async_copy(ufac_hbm.at[ubrow.at[j]], upad.at[dst], sem))
            copies.append(pltpu.async_copy(ifac_hbm.at[ibrow.at[j]], ipad.at[dst], sem))
        for cp in copies:
            cp.wait()

        def group_body(g, carry):
            row0 = g * _L
            uo16 = uoff[pl.ds(h * _HALF + row0, _L)]
            io16 = ioff[pl.ds(h * _HALF + row0, _L)]
            for r in range(_L):
                lr = row0 + r
                bu = uo16[r]
                bi = io16[r]
                acc = upad[lr, pl.ds(bu, _L)] * ipad[lr, pl.ds(bi, _L)]
                for k in range(1, _F // _L):
                    acc = acc + (upad[lr, pl.ds(bu + k * _L, _L)]
                                 * ipad[lr, pl.ds(bi + k * _L, _L)])
                tile[r, ...] = acc
            ssum = plsc.load_gather(tile, [lane, zeros])
            for j in range(1, _L):
                ssum = ssum + plsc.load_gather(tile, [lane, jnp.full((_L,), j, jnp.int32)])
            gr = pl.ds(h * _HALF + row0, _L)
            outv[gr] = ssum + bv[gr]
            return carry

        lax.fori_loop(0, _HALF // _L, group_body, 0)

    pltpu.sync_copy(outv, out_hbm.at[pl.ds(base, _BPW)])


@functools.partial(
    pl.kernel,
    mesh=_mesh,
    out_type=jax.ShapeDtypeStruct((_B,), jnp.float32),
    compiler_params=pltpu.CompilerParams(
        needs_layout_passes=False, use_tc_tiling_on_sc=False
    ),
    scratch_types=[
        pltpu.VMEM((_NCH, _CH), jnp.int32),
        pltpu.VMEM((_NCH, _CH), jnp.int32),
        pltpu.VMEM((_NCH, _CH), jnp.int32),
        pltpu.VMEM((_NCH, _CH), jnp.int32),
        pltpu.VMEM((_BPW,), jnp.int32),
        pltpu.VMEM((_BPW,), jnp.int32),
        pltpu.VMEM((_BPW, _L), jnp.float32),
        pltpu.VMEM((_BPW, _L), jnp.float32),
        pltpu.VMEM((_L,), jnp.float32),
        pltpu.VMEM((_BPW,), jnp.float32),
        pltpu.SemaphoreType.DMA,
    ],
)
def _bias_sc(uids_hbm, iids_hbm, ubias_hbm, ibias_hbm, gb_hbm,
             out_hbm, uidx, iidx, ubidx, ibidx, ulo, ilo, ubrows, ibrows,
             gbv, outv, sem):
    c = lax.axis_index("c")
    s = lax.axis_index("s")
    wid = s * _NC + c
    base = wid * _BPW

    pltpu.sync_copy(uids_hbm.at[pl.ds(wid * _NCH, _NCH)], uidx)
    pltpu.sync_copy(iids_hbm.at[pl.ds(wid * _NCH, _NCH)], iidx)
    pltpu.sync_copy(gb_hbm, gbv)

    for j in range(_NCH):
        for t in range(_CH // _L):
            sl = pl.ds(t * _L, _L)
            fl = pl.ds(j * _CH + t * _L, _L)
            uv = uidx[j, sl]
            iv = iidx[j, sl]
            ubidx[j, sl] = jnp.right_shift(uv, 4)
            ibidx[j, sl] = jnp.right_shift(iv, 4)
            ulo[fl] = jnp.bitwise_and(uv, 15)
            ilo[fl] = jnp.bitwise_and(iv, 15)

    copies = []
    for j in range(_NCH):
        dst = pl.ds(j * _CH, _CH)
        copies.append(pltpu.async_copy(ubias_hbm.at[ubidx.at[j]], ubrows.at[dst], sem))
        copies.append(pltpu.async_copy(ibias_hbm.at[ibidx.at[j]], ibrows.at[dst], sem))
    for cp in copies:
        cp.wait()

    lane = lax.iota(jnp.int32, _L)
    gb = gbv[...]

    def group_body(g, carry):
        row0 = g * _L
        rows16 = row0 + lane
        ub = plsc.load_gather(ubrows, [rows16, ulo[pl.ds(row0, _L)]])
        ib = plsc.load_gather(ibrows, [rows16, ilo[pl.ds(row0, _L)]])
        outv[pl.ds(row0, _L)] = ub + ib + gb
        return carry

    lax.fori_loop(0, _BPW // _L, group_body, 0)

    pltpu.sync_copy(outv, out_hbm.at[pl.ds(base, _BPW)])


_PACKC = 512
_HU = 500224
_HI = 50176


_PACKK = 16


def _pack_body(*refs):
    xa_ref = refs[0]
    o_ref = refs[-1]
    ri = lax.broadcasted_iota(jnp.int32, (_F, _F), 0)
    ci = lax.broadcasted_iota(jnp.int32, (_F, _F), 1)
    ident = (ri == ci).astype(jnp.float32)
    dn = (((0,), (0,)), ((), ()))
    for k in range(_PACKK):
        rows = pl.ds(k * _PACKC, _PACKC)
        o_ref[rows, 0:_F] = lax.dot_general(
            xa_ref[:, pl.ds(k * _PACKC, _PACKC)], ident, dn,
            preferred_element_type=jnp.float32)
        o_ref[rows, _F:2 * _F] = lax.dot_general(
            refs[1 + k][...], ident, dn,
            preferred_element_type=jnp.float32)


def _pack(table_t, half):
    hb = half // _PACKC
    nbmax = table_t.shape[1] // _PACKC
    in_specs = [pl.BlockSpec((_F, _PACKK * _PACKC), lambda j: (0, j))]
    for k in range(_PACKK):
        in_specs.append(pl.BlockSpec(
            (_F, _PACKC),
            lambda j, k=k: (0, jnp.minimum(_PACKK * j + k + hb, nbmax))))
    return pl.pallas_call(
        _pack_body,
        grid=(pl.cdiv(hb, _PACKK),),
        in_specs=in_specs,
        out_specs=pl.BlockSpec((_PACKK * _PACKC, 2 * _F), lambda j: (j, 0)),
        out_shape=jax.ShapeDtypeStruct((half, 2 * _F), jnp.float32),
    )(*([table_t] * (1 + _PACKK)))


def kernel(user_ids, item_ids, user_factors, item_factors, user_bias,
           item_bias, global_bias):
    uids2 = user_ids.reshape(_B // _CH, _CH)
    iids2 = item_ids.reshape(_B // _CH, _CH)
    ub2 = user_bias.reshape(user_bias.shape[0] // _L, _L)
    ib2 = item_bias.reshape(item_bias.shape[0] // _L, _L)
    gb16 = jnp.broadcast_to(global_bias.astype(jnp.float32).reshape(()), (_L,))
    uf2 = _pack(user_factors.T, _HU)
    if2 = _pack(item_factors.T, _HI)
    bsum = _bias_sc(uids2, iids2, ub2, ib2, gb16)
    return _dots_sc(user_ids, item_ids, uf2, if2, bsum)
